# Initial kernel scaffold; baseline (speedup 1.0000x reference)
import jax
import jax.numpy as jnp
from jax.experimental import pallas as pl


def kernel(pred, true):
    raise NotImplementedError("write your pallas kernel here")


# speedup vs baseline: 4.9475x; 4.9475x over previous
import functools

import jax
import jax.numpy as jnp
from jax.experimental import pallas as pl
from jax.experimental.pallas import tpu as pltpu

_EPS = 0.001
_NC = 19


def _body(pred_ref, true_ref, out_ref, hist_ref, *, nsteps):
    step = pl.program_id(0) * pl.num_programs(1) + pl.program_id(1)

    @pl.when(step == 0)
    def _init():
        hist_ref[...] = jnp.zeros_like(hist_ref)

    x = pred_ref[0]
    t = true_ref[0]

    c = x.shape[1]
    cls_iota = jax.lax.broadcasted_iota(jnp.int32, (_NC, c), 0)

    maxv = jnp.max(x, axis=0, keepdims=True)
    cand = jnp.where(x == maxv, cls_iota, _NC)
    p = jnp.min(cand, axis=0, keepdims=True)

    p_oh = (cls_iota == p).astype(jnp.float32)
    t_oh = (cls_iota == t).astype(jnp.float32)

    hist_ref[...] += jax.lax.dot_general(
        p_oh, t_oh, (((1,), (1,)), ((), ())),
        preferred_element_type=jnp.float32)

    @pl.when(step == nsteps - 1)
    def _finish():
        hist = hist_ref[...]
        eye = (jax.lax.broadcasted_iota(jnp.int32, (_NC, _NC), 0)
               == jax.lax.broadcasted_iota(jnp.int32, (_NC, _NC), 1))
        d = jnp.sum(jnp.where(eye, hist, 0.0), axis=1)
        a = jnp.sum(hist, axis=1)
        b = jnp.sum(hist, axis=0)
        dice = 2.0 * d / (a + b + _EPS)
        jac = d / (a + b - d + _EPS)
        loss = jnp.mean(dice) + jnp.mean(jac)
        out_ref[...] = jnp.full((1, 1), 1.0 - loss / 2.0, jnp.float32)


@functools.partial(jax.jit, static_argnames=("chunk",))
def _run(pred, true, chunk=32768):
    bsz, nc, h, w = pred.shape
    npix = h * w
    pred2 = pred.reshape(bsz, nc, npix)
    true2 = true.reshape(bsz, 1, npix).astype(jnp.int32)
    nchunks = npix // chunk
    nsteps = bsz * nchunks

    out = pl.pallas_call(
        functools.partial(_body, nsteps=nsteps),
        grid=(bsz, nchunks),
        in_specs=[
            pl.BlockSpec((1, nc, chunk), lambda i, j: (i, 0, j)),
            pl.BlockSpec((1, 1, chunk), lambda i, j: (i, 0, j)),
        ],
        out_specs=pl.BlockSpec((1, 1), lambda i, j: (0, 0)),
        out_shape=jax.ShapeDtypeStruct((1, 1), jnp.float32),
        scratch_shapes=[pltpu.VMEM((_NC, _NC), jnp.float32)],
    )(pred2, true2)
    return out[0, 0]


def kernel(pred, true):
    return _run(pred, true)


# speedup vs baseline: 5.2015x; 1.0513x over previous
import functools

import jax
import jax.numpy as jnp
from jax.experimental import pallas as pl
from jax.experimental.pallas import tpu as pltpu

_EPS = 0.001
_NC = 19


def _body(pred_ref, true_ref, out_ref, hist_ref, *, nsteps):
    step = pl.program_id(0) * pl.num_programs(1) + pl.program_id(1)

    @pl.when(step == 0)
    def _init():
        hist_ref[...] = jnp.zeros_like(hist_ref)

    x = pred_ref[0]
    t = true_ref[0]

    c = x.shape[1]
    cls_iota = jax.lax.broadcasted_iota(jnp.int32, (_NC, c), 0)

    maxv = jnp.max(x, axis=0, keepdims=True)
    cand = jnp.where(x == maxv, cls_iota, _NC)
    p = jnp.min(cand, axis=0, keepdims=True)

    p_oh = (cls_iota == p).astype(jnp.float32)
    t_oh = (cls_iota == t).astype(jnp.float32)

    hist_ref[...] += jax.lax.dot_general(
        p_oh, t_oh, (((1,), (1,)), ((), ())),
        preferred_element_type=jnp.float32)

    @pl.when(step == nsteps - 1)
    def _finish():
        hist = hist_ref[...]
        eye = (jax.lax.broadcasted_iota(jnp.int32, (_NC, _NC), 0)
               == jax.lax.broadcasted_iota(jnp.int32, (_NC, _NC), 1))
        d = jnp.sum(jnp.where(eye, hist, 0.0), axis=1)
        a = jnp.sum(hist, axis=1)
        b = jnp.sum(hist, axis=0)
        dice = 2.0 * d / (a + b + _EPS)
        jac = d / (a + b - d + _EPS)
        loss = jnp.mean(dice) + jnp.mean(jac)
        out_ref[...] = jnp.full((1, 1), 1.0 - loss / 2.0, jnp.float32)


@functools.partial(jax.jit, static_argnames=("chunk",))
def _run(pred, true, chunk=65536):
    bsz, nc, h, w = pred.shape
    npix = h * w
    pred2 = pred.reshape(bsz, nc, npix)
    true2 = true.reshape(bsz, 1, npix).astype(jnp.int32)
    nchunks = npix // chunk
    nsteps = bsz * nchunks

    out = pl.pallas_call(
        functools.partial(_body, nsteps=nsteps),
        grid=(bsz, nchunks),
        in_specs=[
            pl.BlockSpec((1, nc, chunk), lambda i, j: (i, 0, j)),
            pl.BlockSpec((1, 1, chunk), lambda i, j: (i, 0, j)),
        ],
        out_specs=pl.BlockSpec((1, 1), lambda i, j: (0, 0)),
        out_shape=jax.ShapeDtypeStruct((1, 1), jnp.float32),
        scratch_shapes=[pltpu.VMEM((_NC, _NC), jnp.float32)],
    )(pred2, true2)
    return out[0, 0]


def kernel(pred, true):
    return _run(pred, true)


# speedup vs baseline: 5.2783x; 1.0148x over previous
import functools

import jax
import jax.numpy as jnp
from jax.experimental import pallas as pl
from jax.experimental.pallas import tpu as pltpu

_EPS = 0.001
_NC = 19


def _body(pred_ref, true_ref, out_ref, hist_ref, *, nsteps):
    step = pl.program_id(0) * pl.num_programs(1) + pl.program_id(1)

    @pl.when(step == 0)
    def _init():
        hist_ref[...] = jnp.zeros_like(hist_ref)

    x = pred_ref[0]
    t = true_ref[0]

    c = x.shape[1]
    cls_iota = jax.lax.broadcasted_iota(jnp.int32, (_NC, 1), 0)

    maxv = jnp.max(x, axis=0, keepdims=True)
    eq = (x == maxv).astype(jnp.float32)

    w = (2.0 ** (-cls_iota.astype(jnp.float32))).reshape(1, _NC)
    m = jax.lax.dot_general(w, eq, (((1,), (0,)), ((), ())),
                            preferred_element_type=jnp.float32)
    thr = 2.0 ** (1 - cls_iota.astype(jnp.float32))
    first = (m < thr).astype(jnp.float32)
    p_oh = eq * first

    t_oh = (cls_iota == t).astype(jnp.float32)

    hist_ref[...] += jax.lax.dot_general(
        p_oh, t_oh, (((1,), (1,)), ((), ())),
        preferred_element_type=jnp.float32)

    @pl.when(step == nsteps - 1)
    def _finish():
        hist = hist_ref[...]
        eye = (jax.lax.broadcasted_iota(jnp.int32, (_NC, _NC), 0)
               == jax.lax.broadcasted_iota(jnp.int32, (_NC, _NC), 1))
        d = jnp.sum(jnp.where(eye, hist, 0.0), axis=1)
        a = jnp.sum(hist, axis=1)
        b = jnp.sum(hist, axis=0)
        dice = 2.0 * d / (a + b + _EPS)
        jac = d / (a + b - d + _EPS)
        loss = jnp.mean(dice) + jnp.mean(jac)
        out_ref[...] = jnp.full((1, 1), 1.0 - loss / 2.0, jnp.float32)


@functools.partial(jax.jit, static_argnames=("chunk",))
def _run(pred, true, chunk=65536):
    bsz, nc, h, w = pred.shape
    npix = h * w
    pred2 = pred.reshape(bsz, nc, npix)
    true2 = true.reshape(bsz, 1, npix).astype(jnp.int32)
    nchunks = npix // chunk
    nsteps = bsz * nchunks

    out = pl.pallas_call(
        functools.partial(_body, nsteps=nsteps),
        grid=(bsz, nchunks),
        in_specs=[
            pl.BlockSpec((1, nc, chunk), lambda i, j: (i, 0, j)),
            pl.BlockSpec((1, 1, chunk), lambda i, j: (i, 0, j)),
        ],
        out_specs=pl.BlockSpec((1, 1), lambda i, j: (0, 0)),
        out_shape=jax.ShapeDtypeStruct((1, 1), jnp.float32),
        scratch_shapes=[pltpu.VMEM((_NC, _NC), jnp.float32)],
    )(pred2, true2)
    return out[0, 0]


def kernel(pred, true):
    return _run(pred, true)
